# Initial kernel scaffold; baseline (speedup 1.0000x reference)
import jax
import jax.numpy as jnp
from jax.experimental import pallas as pl


def kernel(x, edge_index, batch, W_emb, b_emb, W_first, b_first, W_l, b_l, W_r, W_ih, W_hh, b_ih, b_hh, lW_ih, lW_hh, lb_ih, lb_hh, W_lin, b_lin, W_fin, b_fin):
    raise NotImplementedError("write your pallas kernel here")


# speedup vs baseline: 3.5638x; 3.5638x over previous
import functools

import jax
import jax.numpy as jnp
from jax import lax
from jax.experimental import pallas as pl
from jax.experimental.pallas import tpu as pltpu
from jax.experimental.pallas import tpu_sc as plsc

N = 10000
E = 320000
D = 128
ENC = 128
LAT = 64
G = 128
STEPS = 3

NC = 2
NS = 16
CHUNK = 128
CPT = 80
EPT = CPT * CHUNK
E_PAD = NC * NS * EPT
N_ACC = N + 112
RPT = N_ACC // NS

ROWS_BLK = 1000


def _segsum_sc(h, src_p, dst_p, zeros_acc):
  mesh = plsc.VectorSubcoreMesh(core_axis_name="c", subcore_axis_name="s")

  @functools.partial(
      pl.kernel,
      mesh=mesh,
      out_type=(
          jax.ShapeDtypeStruct((N_ACC, D), jnp.float32),
          jax.ShapeDtypeStruct((N_ACC, D), jnp.float32),
      ),
      scratch_types=[
          pltpu.VMEM((CPT, CHUNK), jnp.int32),
          pltpu.VMEM((CPT, CHUNK), jnp.int32),
          pltpu.VMEM((CHUNK, D), jnp.float32),
          pltpu.VMEM_SHARED((N_ACC, D), jnp.float32),
          pltpu.SemaphoreType.DMA,
      ],
  )
  def seg_kernel(h_hbm, src_hbm, dst_hbm, z_hbm, agg0_hbm, agg1_hbm,
                 src_v, dst_v, rows_v, acc, sem):
    c = lax.axis_index("c")
    s = lax.axis_index("s")
    sl = pl.ds(s * RPT, RPT)
    pltpu.sync_copy(z_hbm.at[sl], acc.at[sl])
    pltpu.sync_copy(src_hbm.at[c, s], src_v)
    pltpu.sync_copy(dst_hbm.at[c, s], dst_v)
    plsc.subcore_barrier()

    def body(j, carry):
      pltpu.async_copy(h_hbm.at[src_v.at[j]], rows_v, sem).wait()
      pltpu.sync_copy(rows_v, acc.at[dst_v.at[j]], add=True)
      return carry

    lax.fori_loop(0, CPT, body, 0)
    plsc.subcore_barrier()

    @pl.when(c == 0)
    def _():
      pltpu.sync_copy(acc.at[sl], agg0_hbm.at[sl])

    @pl.when(c == 1)
    def _():
      pltpu.sync_copy(acc.at[sl], agg1_hbm.at[sl])

  return seg_kernel(h, src_p, dst_p, zeros_acc)


def _lrelu(t):
  return jnp.where(t >= 0, t, 0.01 * t)


def _embed_body(x_ref, we_ref, be_ref, wf_ref, bf_ref, o_ref):
  f32 = jnp.float32
  t = jnp.dot(x_ref[...], we_ref[...], preferred_element_type=f32) + be_ref[...]
  t = _lrelu(t)
  t = jnp.dot(t, wf_ref[...], preferred_element_type=f32) + bf_ref[...]
  o_ref[...] = _lrelu(t)


def _embed_tc(x, W_emb, b_emb, W_first, b_first):
  nb = N // ROWS_BLK
  return pl.pallas_call(
      _embed_body,
      grid=(nb,),
      in_specs=[
          pl.BlockSpec((ROWS_BLK, D), lambda j: (j, 0)),
          pl.BlockSpec((D, ENC), lambda j: (0, 0)),
          pl.BlockSpec((1, ENC), lambda j: (0, 0)),
          pl.BlockSpec((ENC, ENC), lambda j: (0, 0)),
          pl.BlockSpec((1, ENC), lambda j: (0, 0)),
      ],
      out_specs=pl.BlockSpec((ROWS_BLK, ENC), lambda j: (j, 0)),
      out_shape=jax.ShapeDtypeStruct((N, ENC), jnp.float32),
  )(x, W_emb, b_emb, W_first, b_first)


def _gru_body(a0_ref, a1_ref, h_ref, wl_ref, bl_ref, wr_ref, wih_ref,
              whh_ref, bih_ref, bhh_ref, o_ref):
  f32 = jnp.float32
  dn_t = (((1,), (1,)), ((), ()))
  agg = a0_ref[...] + a1_ref[...]
  h = h_ref[...]
  m = (jnp.dot(agg, wl_ref[...], preferred_element_type=f32) + bl_ref[...]
       + jnp.dot(h, wr_ref[...], preferred_element_type=f32))
  m = jnp.maximum(m, 0.0)
  gi = lax.dot_general(m, wih_ref[...], dn_t, preferred_element_type=f32) + bih_ref[...]
  gh = lax.dot_general(h, whh_ref[...], dn_t, preferred_element_type=f32) + bhh_ref[...]
  r = jax.nn.sigmoid(gi[:, 0:ENC] + gh[:, 0:ENC])
  z = jax.nn.sigmoid(gi[:, ENC:2 * ENC] + gh[:, ENC:2 * ENC])
  n = jnp.tanh(gi[:, 2 * ENC:3 * ENC] + r * gh[:, 2 * ENC:3 * ENC])
  o_ref[...] = (1.0 - z) * n + z * h


def _gru_tc(agg0, agg1, h, W_l, b_l, W_r, W_ih, W_hh, b_ih, b_hh):
  nb = N // ROWS_BLK
  return pl.pallas_call(
      _gru_body,
      grid=(nb,),
      in_specs=[
          pl.BlockSpec((ROWS_BLK, ENC), lambda j: (j, 0)),
          pl.BlockSpec((ROWS_BLK, ENC), lambda j: (j, 0)),
          pl.BlockSpec((ROWS_BLK, ENC), lambda j: (j, 0)),
          pl.BlockSpec((ENC, ENC), lambda j: (0, 0)),
          pl.BlockSpec((1, ENC), lambda j: (0, 0)),
          pl.BlockSpec((ENC, ENC), lambda j: (0, 0)),
          pl.BlockSpec((3 * ENC, ENC), lambda j: (0, 0)),
          pl.BlockSpec((3 * ENC, ENC), lambda j: (0, 0)),
          pl.BlockSpec((1, 3 * ENC), lambda j: (0, 0)),
          pl.BlockSpec((1, 3 * ENC), lambda j: (0, 0)),
      ],
      out_specs=pl.BlockSpec((ROWS_BLK, ENC), lambda j: (j, 0)),
      out_shape=jax.ShapeDtypeStruct((N, ENC), jnp.float32),
  )(agg0, agg1, h, W_l, b_l, W_r, W_ih, W_hh, b_ih, b_hh)


def _s2s_body(h_ref, b_ref, lwih_ref, lwhh_ref, lbih_ref, lbhh_ref,
              wlin_ref, blin_ref, wfin_ref, bfin_ref, o_ref):
  f32 = jnp.float32
  dn_t = (((1,), (1,)), ((), ()))
  dn_c = (((0,), (0,)), ((), ()))
  out = h_ref[...]
  bvec = b_ref[...]
  gids = lax.broadcasted_iota(jnp.int32, (N, G), 1)
  onehot = (bvec == gids).astype(f32)
  q_star = jnp.zeros((G, 2 * ENC), f32)
  hl = jnp.zeros((G, ENC), f32)
  cl = jnp.zeros((G, ENC), f32)
  neg = jnp.float32(-1e30)
  for _ in range(3):
    gates = (lax.dot_general(q_star, lwih_ref[...], dn_t, preferred_element_type=f32)
             + lbih_ref[...]
             + lax.dot_general(hl, lwhh_ref[...], dn_t, preferred_element_type=f32)
             + lbhh_ref[...])
    gi = jax.nn.sigmoid(gates[:, 0:ENC])
    gf = jax.nn.sigmoid(gates[:, ENC:2 * ENC])
    gg = jnp.tanh(gates[:, 2 * ENC:3 * ENC])
    go = jax.nn.sigmoid(gates[:, 3 * ENC:4 * ENC])
    cl = gf * cl + gi * gg
    hl = go * jnp.tanh(cl)
    hlb = jnp.dot(onehot, hl, preferred_element_type=f32)
    e = jnp.sum(out * hlb, axis=1, keepdims=True)
    em = jnp.where(onehot > 0, e, neg)
    emax = jnp.max(em, axis=0, keepdims=True)
    emax = jnp.where(emax <= neg, 0.0, emax)
    eb = jnp.sum(onehot * emax, axis=1, keepdims=True)
    a = jnp.exp(e - eb)
    denom = jnp.sum(onehot * a, axis=0, keepdims=True)
    db = jnp.sum(onehot * denom, axis=1, keepdims=True)
    a = a / (db + 1e-16)
    rvec = lax.dot_general(onehot * a, out, dn_c,
                           preferred_element_type=f32)
    q_star = jnp.concatenate([hl, rvec], axis=1)
  out2 = jnp.dot(q_star, wlin_ref[...], preferred_element_type=f32) + blin_ref[...]
  out2 = _lrelu(out2)
  o_ref[...] = jnp.dot(out2, wfin_ref[...], preferred_element_type=f32) + bfin_ref[...]


def _s2s_tc(h, batch2d, lW_ih, lW_hh, lb_ih, lb_hh, W_lin, b_lin, W_fin, b_fin):
  return pl.pallas_call(
      _s2s_body,
      out_shape=jax.ShapeDtypeStruct((G, LAT), jnp.float32),
  )(h, batch2d, lW_ih, lW_hh, lb_ih, lb_hh, W_lin, b_lin, W_fin, b_fin)


def kernel(x, edge_index, batch, W_emb, b_emb, W_first, b_first, W_l, b_l,
           W_r, W_ih, W_hh, b_ih, b_hh, lW_ih, lW_hh, lb_ih, lb_hh, W_lin,
           b_lin, W_fin, b_fin):
  i32 = jnp.int32
  src = edge_index[0].astype(i32)
  dst = edge_index[1].astype(i32)
  pad = E_PAD - E
  src_p = jnp.concatenate([src, jnp.zeros((pad,), i32)]).reshape(NC, NS, CPT, CHUNK)
  dst_p = jnp.concatenate([dst, jnp.full((pad,), N, i32)]).reshape(NC, NS, CPT, CHUNK)
  zeros_acc = jnp.zeros((N_ACC, D), jnp.float32)
  r2 = lambda b: b.reshape(1, -1)

  h = _embed_tc(x, W_emb, r2(b_emb), W_first, r2(b_first))
  for _ in range(STEPS):
    agg0, agg1 = _segsum_sc(h, src_p, dst_p, zeros_acc)
    h = _gru_tc(agg0[:N], agg1[:N], h, W_l, r2(b_l), W_r, W_ih, W_hh,
                r2(b_ih), r2(b_hh))
  return _s2s_tc(h, batch.astype(i32).reshape(N, 1), lW_ih, lW_hh,
                 r2(lb_ih), r2(lb_hh), W_lin, r2(b_lin), W_fin, r2(b_fin))
